# baseline (speedup 1.0000x reference)
import functools

import jax
import jax.numpy as jnp
from jax import lax
from jax.experimental import pallas as pl
from jax.experimental.pallas import tpu as pltpu
from jax.experimental.pallas import tpu_sc as plsc

N = 320000
D = 128
S = 10000
NC = 2
NS = 16
NW = NC * NS
C = 128
NBLK = N // C
GLO = NBLK // NW
NHI = NBLK - GLO * NW
NB = 2
SPT = S // NS
CW = 16


def _sc_body(x_hbm, ids_hbm, sums_hbm, cnt_hbm,
             acc_sh, cnt_sh, rows_v, ids_v, ones_v, zcnt_v, *sems):
    c = lax.axis_index("c")
    s = lax.axis_index("s")
    wid = c * NS + s
    base = GLO * wid + jnp.maximum(wid - (NW - NHI), 0)
    ng = GLO + (wid >= NW - NHI).astype(jnp.int32)

    gsems = sems[0:NB]
    isems = sems[NB:2 * NB]
    ssems = sems[2 * NB:3 * NB]

    def start(g, b):
        pltpu.async_copy(x_hbm.at[pl.ds((base + g) * C, C)], rows_v.at[b],
                         gsems[b])
        pltpu.async_copy(ids_hbm.at[pl.ds(base + g, 1)], ids_v.at[b],
                         isems[b])

    def wait(g, b):
        pltpu.make_async_copy(x_hbm.at[pl.ds((base + g) * C, C)],
                              rows_v.at[b], gsems[b]).wait()
        pltpu.make_async_copy(ids_hbm.at[pl.ds(base + g, 1)], ids_v.at[b],
                              isems[b]).wait()

    def fire(b):
        idx = ids_v.at[b, 0]
        pltpu.async_copy(rows_v.at[b], acc_sh.at[idx], ssems[b], add=True)
        pltpu.async_copy(ones_v, cnt_sh.at[idx], ssems[b], add=True)

    def wait_scat(b):
        idx = ids_v.at[b, 0]
        pltpu.make_async_copy(rows_v.at[b], acc_sh.at[idx], ssems[b]).wait()
        pltpu.make_async_copy(ones_v, cnt_sh.at[idx], ssems[b]).wait()

    start(0, 0)

    def zrow(r, carry):
        for k in range(D // 16):
            rows_v[1, r, pl.ds(k * 16, 16)] = jnp.zeros((16,), jnp.float32)
        ones_v[r, :] = jnp.full((16,), 1.0, jnp.float32)
        zcnt_v[r, :] = jnp.zeros((16,), jnp.float32)
        return carry
    lax.fori_loop(0, C, zrow, 0)

    for j in range(SPT // C):
        off = s * SPT + j * C
        pltpu.sync_copy(rows_v.at[1], acc_sh.at[pl.ds(off, C)])
        pltpu.sync_copy(zcnt_v, cnt_sh.at[pl.ds(off, C)])
    rem = SPT % C
    if rem:
        off = s * SPT + (SPT // C) * C
        pltpu.sync_copy(rows_v.at[1, pl.ds(0, rem)],
                        acc_sh.at[pl.ds(off, rem)])
        pltpu.sync_copy(zcnt_v.at[pl.ds(0, rem)], cnt_sh.at[pl.ds(off, rem)])
    plsc.subcore_barrier()

    start(1, 1)

    def step(t, carry):
        for b in range(NB):
            g = t * NB + b
            wait(g, b)
            fire(b)
            wait_scat(b)

            @pl.when(g + 2 < ng)
            def _():
                start(g + 2, b)
        return carry
    lax.fori_loop(0, GLO // NB, step, 0)

    @pl.when(ng > GLO)
    def _():
        wait(GLO, 0)
        fire(0)
        wait_scat(0)

    plsc.subcore_barrier()

    pltpu.sync_copy(acc_sh.at[pl.ds(s * SPT, SPT)],
                    sums_hbm.at[c, pl.ds(s * SPT, SPT)])
    pltpu.sync_copy(cnt_sh.at[pl.ds(s * SPT, SPT)],
                    cnt_hbm.at[c, pl.ds(s * SPT, SPT)])


_phase1 = functools.partial(
    pl.kernel,
    out_type=(jax.ShapeDtypeStruct((NC, S, D), jnp.float32),
              jax.ShapeDtypeStruct((NC, S, CW), jnp.float32)),
    mesh=plsc.VectorSubcoreMesh(core_axis_name="c", subcore_axis_name="s",
                                num_cores=NC, num_subcores=NS),
    scratch_types=[
        pltpu.VMEM_SHARED((S, D), jnp.float32),
        pltpu.VMEM_SHARED((S, CW), jnp.float32),
        pltpu.VMEM((NB, C, D), jnp.float32),
        pltpu.VMEM((NB, 1, C), jnp.int32),
        pltpu.VMEM((C, CW), jnp.float32),
        pltpu.VMEM((C, CW), jnp.float32),
    ] + [pltpu.SemaphoreType.DMA] * (3 * NB),
    compiler_params=pltpu.CompilerParams(use_tc_tiling_on_sc=False),
)(_sc_body)


SZ = 320


def _div_body(sums_hbm, cnt_hbm, out_hbm, sa, sb, ca, cb, dsem):
    c = lax.axis_index("c")
    s = lax.axis_index("s")
    wid = c * NS + s
    base = wid * SZ

    def run(sz):
        pltpu.async_copy(sums_hbm.at[0, pl.ds(base, sz)],
                         sa.at[pl.ds(0, sz)], dsem)
        pltpu.async_copy(sums_hbm.at[1, pl.ds(base, sz)],
                         sb.at[pl.ds(0, sz)], dsem)
        pltpu.async_copy(cnt_hbm.at[0, pl.ds(base, sz)],
                         ca.at[pl.ds(0, sz)], dsem)
        pltpu.async_copy(cnt_hbm.at[1, pl.ds(base, sz)],
                         cb.at[pl.ds(0, sz)], dsem)
        pltpu.make_async_copy(sums_hbm.at[0, pl.ds(base, sz)],
                              sa.at[pl.ds(0, sz)], dsem).wait()
        pltpu.make_async_copy(sums_hbm.at[1, pl.ds(base, sz)],
                              sb.at[pl.ds(0, sz)], dsem).wait()
        pltpu.make_async_copy(cnt_hbm.at[0, pl.ds(base, sz)],
                              ca.at[pl.ds(0, sz)], dsem).wait()
        pltpu.make_async_copy(cnt_hbm.at[1, pl.ds(base, sz)],
                              cb.at[pl.ds(0, sz)], dsem).wait()

        def row(r, carry):
            cv = ca[r, :] + cb[r, :]
            rec = 1.0 / jnp.maximum(cv, 1.0)
            for k in range(D // 16):
                col = pl.ds(k * 16, 16)
                sa[r, col] = (sa[r, col] + sb[r, col]) * rec
            return carry
        lax.fori_loop(0, sz, row, 0)
        pltpu.sync_copy(sa.at[pl.ds(0, sz)], out_hbm.at[pl.ds(base, sz)])

    @pl.when(wid < NW - 1)
    def _():
        run(SZ)

    @pl.when(wid == NW - 1)
    def _():
        run(S - (NW - 1) * SZ)


_phase2 = functools.partial(
    pl.kernel,
    out_type=jax.ShapeDtypeStruct((S, D), jnp.float32),
    mesh=plsc.VectorSubcoreMesh(core_axis_name="c", subcore_axis_name="s",
                                num_cores=NC, num_subcores=NS),
    scratch_types=[
        pltpu.VMEM((SZ, D), jnp.float32),
        pltpu.VMEM((SZ, D), jnp.float32),
        pltpu.VMEM((SZ, CW), jnp.float32),
        pltpu.VMEM((SZ, CW), jnp.float32),
        pltpu.SemaphoreType.DMA,
    ],
    compiler_params=pltpu.CompilerParams(use_tc_tiling_on_sc=False),
)(_div_body)


def kernel(x, segment_ids, num_segments):
    del num_segments
    ids2d = segment_ids.astype(jnp.int32).reshape(NBLK, C)
    sums, counts = _phase1(x, ids2d)
    return _phase2(sums, counts)


# speedup vs baseline: 1.0601x; 1.0601x over previous
import functools

import jax
import jax.numpy as jnp
from jax import lax
from jax.experimental import pallas as pl
from jax.experimental.pallas import tpu as pltpu
from jax.experimental.pallas import tpu_sc as plsc

N = 320000
D = 128
S = 10000
NC = 2
NS = 16
NW = NC * NS
C = 128
NBLK = N // C
GLO = NBLK // NW
NHI = NBLK - GLO * NW
NB = 2
SPT = S // NS
CW = 16


def _sc_body(x_hbm, ids_hbm, sums_hbm, cnt_hbm,
             acc_sh, cnt_sh, rows_v, ids_v, ones_v, zcnt_v, *sems):
    c = lax.axis_index("c")
    s = lax.axis_index("s")
    wid = c * NS + s
    base = GLO * wid + jnp.maximum(wid - (NW - NHI), 0)
    ng = GLO + (wid >= NW - NHI).astype(jnp.int32)

    gsems = sems[0:NB]
    isems = sems[NB:2 * NB]
    ssems = sems[2 * NB:3 * NB]

    def start(g, b):
        pltpu.async_copy(x_hbm.at[pl.ds((base + g) * C, C)], rows_v.at[b],
                         gsems[b])
        pltpu.async_copy(ids_hbm.at[pl.ds(base + g, 1)], ids_v.at[b],
                         isems[b])

    def wait(g, b):
        pltpu.make_async_copy(x_hbm.at[pl.ds((base + g) * C, C)],
                              rows_v.at[b], gsems[b]).wait()
        pltpu.make_async_copy(ids_hbm.at[pl.ds(base + g, 1)], ids_v.at[b],
                              isems[b]).wait()

    def fire(b):
        idx = ids_v.at[b, 0]
        pltpu.async_copy(rows_v.at[b], acc_sh.at[idx], ssems[b], add=True)
        pltpu.async_copy(ones_v, cnt_sh.at[idx], ssems[b], add=True)

    def wait_scat(b):
        idx = ids_v.at[b, 0]
        pltpu.make_async_copy(rows_v.at[b], acc_sh.at[idx], ssems[b]).wait()
        pltpu.make_async_copy(ones_v, cnt_sh.at[idx], ssems[b]).wait()

    start(0, 0)

    def zrow(r, carry):
        for k in range(D // 16):
            rows_v[1, r, pl.ds(k * 16, 16)] = jnp.zeros((16,), jnp.float32)
        ones_v[r, :] = jnp.full((16,), 1.0, jnp.float32)
        zcnt_v[r, :] = jnp.zeros((16,), jnp.float32)
        return carry
    lax.fori_loop(0, C, zrow, 0)

    for j in range(SPT // C):
        off = s * SPT + j * C
        pltpu.sync_copy(rows_v.at[1], acc_sh.at[pl.ds(off, C)])
        pltpu.sync_copy(zcnt_v, cnt_sh.at[pl.ds(off, C)])
    rem = SPT % C
    if rem:
        off = s * SPT + (SPT // C) * C
        pltpu.sync_copy(rows_v.at[1, pl.ds(0, rem)],
                        acc_sh.at[pl.ds(off, rem)])
        pltpu.sync_copy(zcnt_v.at[pl.ds(0, rem)], cnt_sh.at[pl.ds(off, rem)])
    plsc.subcore_barrier()

    start(1, 1)

    def step(t, carry):
        for b in range(NB):
            g = t * NB + b
            wait(g, b)
            fire(b)
            wait_scat(b)

            @pl.when(g + 2 < ng)
            def _():
                start(g + 2, b)
        return carry
    lax.fori_loop(0, GLO // NB, step, 0)

    @pl.when(ng > GLO)
    def _():
        wait(GLO, 0)
        fire(0)
        wait_scat(0)

    plsc.subcore_barrier()

    pltpu.sync_copy(acc_sh.at[pl.ds(s * SPT, SPT)],
                    sums_hbm.at[c, pl.ds(s * SPT, SPT)])
    pltpu.sync_copy(cnt_sh.at[pl.ds(s * SPT, SPT)],
                    cnt_hbm.at[c, pl.ds(s * SPT, SPT)])


_phase1 = functools.partial(
    pl.kernel,
    out_type=(jax.ShapeDtypeStruct((NC, S, D), jnp.float32),
              jax.ShapeDtypeStruct((NC, S, CW), jnp.float32)),
    mesh=plsc.VectorSubcoreMesh(core_axis_name="c", subcore_axis_name="s",
                                num_cores=NC, num_subcores=NS),
    scratch_types=[
        pltpu.VMEM_SHARED((S, D), jnp.float32),
        pltpu.VMEM_SHARED((S, CW), jnp.float32),
        pltpu.VMEM((NB, C, D), jnp.float32),
        pltpu.VMEM((NB, 1, C), jnp.int32),
        pltpu.VMEM((C, CW), jnp.float32),
        pltpu.VMEM((C, CW), jnp.float32),
    ] + [pltpu.SemaphoreType.DMA] * (3 * NB),
    compiler_params=pltpu.CompilerParams(use_tc_tiling_on_sc=False),
)(_sc_body)


BS = 2000


def _div_body(sums_ref, cnt_ref, out_ref):
    sm = sums_ref[0] + sums_ref[1]
    ct = cnt_ref[0, :, :1] + cnt_ref[1, :, :1]
    out_ref[...] = sm / jnp.maximum(ct, 1.0)


def _phase2(sums, counts):
    return pl.pallas_call(
        _div_body,
        grid=(S // BS,),
        in_specs=[
            pl.BlockSpec((NC, BS, D), lambda i: (0, i, 0)),
            pl.BlockSpec((NC, BS, CW), lambda i: (0, i, 0)),
        ],
        out_specs=pl.BlockSpec((BS, D), lambda i: (i, 0)),
        out_shape=jax.ShapeDtypeStruct((S, D), jnp.float32),
    )(sums, counts)


def kernel(x, segment_ids, num_segments):
    del num_segments
    ids2d = segment_ids.astype(jnp.int32).reshape(NBLK, C)
    sums, counts = _phase1(x, ids2d)
    return _phase2(sums, counts)


# speedup vs baseline: 1.0657x; 1.0053x over previous
import functools

import jax
import jax.numpy as jnp
from jax import lax
from jax.experimental import pallas as pl
from jax.experimental.pallas import tpu as pltpu
from jax.experimental.pallas import tpu_sc as plsc

N = 320000
D = 128
S = 10000
NC = 2
NS = 16
NW = NC * NS
C = 128
NBLK = N // C
GLO = NBLK // NW
NHI = NBLK - GLO * NW
NB = 2
SPT = S // NS
CW = 16


def _sc_body(x_hbm, ids_hbm, sums_hbm, cnt_hbm,
             acc_sh, cnt_sh, rows_v, ids_v, ones_v, zcnt_v, *sems):
    c = lax.axis_index("c")
    s = lax.axis_index("s")
    wid = c * NS + s
    base = GLO * wid + jnp.maximum(wid - (NW - NHI), 0)
    ng = GLO + (wid >= NW - NHI).astype(jnp.int32)

    gsems = sems[0:NB]
    isems = sems[NB:2 * NB]
    ssems = sems[2 * NB:3 * NB]

    def start(g, b):
        pltpu.async_copy(x_hbm.at[pl.ds((base + g) * C, C)], rows_v.at[b],
                         gsems[b])
        pltpu.async_copy(ids_hbm.at[pl.ds(base + g, 1)], ids_v.at[b],
                         isems[b])

    def wait(g, b):
        pltpu.make_async_copy(x_hbm.at[pl.ds((base + g) * C, C)],
                              rows_v.at[b], gsems[b]).wait()
        pltpu.make_async_copy(ids_hbm.at[pl.ds(base + g, 1)], ids_v.at[b],
                              isems[b]).wait()

    def fire(b):
        idx = ids_v.at[b, 0]
        pltpu.async_copy(rows_v.at[b], acc_sh.at[idx], ssems[b], add=True)
        pltpu.async_copy(ones_v, cnt_sh.at[idx], ssems[b], add=True)

    def wait_scat(b):
        idx = ids_v.at[b, 0]
        pltpu.make_async_copy(rows_v.at[b], acc_sh.at[idx], ssems[b]).wait()
        pltpu.make_async_copy(ones_v, cnt_sh.at[idx], ssems[b]).wait()

    start(0, 0)

    def zrow(r, carry):
        for k in range(D // 16):
            rows_v[1, r, pl.ds(k * 16, 16)] = jnp.zeros((16,), jnp.float32)
        ones_v[r, :] = jnp.full((16,), 1.0, jnp.float32)
        zcnt_v[r, :] = jnp.zeros((16,), jnp.float32)
        return carry
    lax.fori_loop(0, C, zrow, 0)

    for j in range(SPT // C):
        off = s * SPT + j * C
        pltpu.sync_copy(rows_v.at[1], acc_sh.at[pl.ds(off, C)])
        pltpu.sync_copy(zcnt_v, cnt_sh.at[pl.ds(off, C)])
    rem = SPT % C
    if rem:
        off = s * SPT + (SPT // C) * C
        pltpu.sync_copy(rows_v.at[1, pl.ds(0, rem)],
                        acc_sh.at[pl.ds(off, rem)])
        pltpu.sync_copy(zcnt_v.at[pl.ds(0, rem)], cnt_sh.at[pl.ds(off, rem)])
    plsc.subcore_barrier()

    start(1, 1)

    def step(t, carry):
        for b in range(NB):
            g = t * NB + b
            wait(g, b)
            fire(b)
            wait_scat(b)

            @pl.when(g + 2 < ng)
            def _():
                start(g + 2, b)
        return carry
    lax.fori_loop(0, GLO // NB, step, 0)

    @pl.when(ng > GLO)
    def _():
        wait(GLO, 0)
        fire(0)
        wait_scat(0)

    plsc.subcore_barrier()

    pltpu.sync_copy(acc_sh.at[pl.ds(s * SPT, SPT)],
                    sums_hbm.at[c, pl.ds(s * SPT, SPT)])
    pltpu.sync_copy(cnt_sh.at[pl.ds(s * SPT, SPT)],
                    cnt_hbm.at[c, pl.ds(s * SPT, SPT)])


_phase1 = functools.partial(
    pl.kernel,
    out_type=(jax.ShapeDtypeStruct((NC, S, D), jnp.float32),
              jax.ShapeDtypeStruct((NC, S, CW), jnp.float32)),
    mesh=plsc.VectorSubcoreMesh(core_axis_name="c", subcore_axis_name="s",
                                num_cores=NC, num_subcores=NS),
    scratch_types=[
        pltpu.VMEM_SHARED((S, D), jnp.float32),
        pltpu.VMEM_SHARED((S, CW), jnp.float32),
        pltpu.VMEM((NB, C, D), jnp.float32),
        pltpu.VMEM((NB, 1, C), jnp.int32),
        pltpu.VMEM((C, CW), jnp.float32),
        pltpu.VMEM((C, CW), jnp.float32),
    ] + [pltpu.SemaphoreType.DMA] * (3 * NB),
    compiler_params=pltpu.CompilerParams(use_tc_tiling_on_sc=False),
)(_sc_body)


BS = 5000


def _div_body(sums_ref, cnt_ref, out_ref):
    sm = sums_ref[0] + sums_ref[1]
    ct = cnt_ref[0, :, :1] + cnt_ref[1, :, :1]
    out_ref[...] = sm / jnp.maximum(ct, 1.0)


def _phase2(sums, counts):
    return pl.pallas_call(
        _div_body,
        grid=(S // BS,),
        in_specs=[
            pl.BlockSpec((NC, BS, D), lambda i: (0, i, 0)),
            pl.BlockSpec((NC, BS, CW), lambda i: (0, i, 0)),
        ],
        out_specs=pl.BlockSpec((BS, D), lambda i: (i, 0)),
        out_shape=jax.ShapeDtypeStruct((S, D), jnp.float32),
    )(sums, counts)


def kernel(x, segment_ids, num_segments):
    del num_segments
    ids2d = segment_ids.astype(jnp.int32).reshape(NBLK, C)
    sums, counts = _phase1(x, ids2d)
    return _phase2(sums, counts)


# speedup vs baseline: 1.0690x; 1.0031x over previous
import functools

import jax
import jax.numpy as jnp
from jax import lax
from jax.experimental import pallas as pl
from jax.experimental.pallas import tpu as pltpu
from jax.experimental.pallas import tpu_sc as plsc

N = 320000
D = 128
S = 10000
NC = 2
NS = 16
NW = NC * NS
C = 128
NBLK = N // C
GLO = NBLK // NW
NHI = NBLK - GLO * NW
NB = 2
SPT = S // NS
CW = 16


def _sc_body(x_hbm, ids_hbm, sums_hbm, cnt_hbm,
             acc_sh, cnt_sh, rows_v, ids_v, ones_v, zcnt_v, *sems):
    c = lax.axis_index("c")
    s = lax.axis_index("s")
    wid = c * NS + s
    base = GLO * wid + jnp.maximum(wid - (NW - NHI), 0)
    ng = GLO + (wid >= NW - NHI).astype(jnp.int32)

    gsems = sems[0:NB]
    isems = sems[NB:2 * NB]
    ssems = sems[2 * NB:3 * NB]
    osem = sems[3 * NB]

    def start(g, b):
        pltpu.async_copy(x_hbm.at[pl.ds((base + g) * C, C)], rows_v.at[b],
                         gsems[b])
        pltpu.async_copy(ids_hbm.at[pl.ds(base + g, 1)], ids_v.at[b],
                         isems[b])

    def wait(g, b):
        pltpu.make_async_copy(x_hbm.at[pl.ds((base + g) * C, C)],
                              rows_v.at[b], gsems[b]).wait()
        pltpu.make_async_copy(ids_hbm.at[pl.ds(base + g, 1)], ids_v.at[b],
                              isems[b]).wait()

    def fire(b):
        idx = ids_v.at[b, 0]
        pltpu.async_copy(rows_v.at[b], acc_sh.at[idx], ssems[b], add=True)
        pltpu.async_copy(ones_v, cnt_sh.at[idx], osem, add=True)

    def wait_scat(b):
        idx = ids_v.at[b, 0]
        pltpu.make_async_copy(rows_v.at[b], acc_sh.at[idx], ssems[b]).wait()

    def wait_ones(b):
        idx = ids_v.at[b, 0]
        pltpu.make_async_copy(ones_v, cnt_sh.at[idx], osem).wait()

    start(0, 0)

    def zrow(r, carry):
        for k in range(D // 16):
            rows_v[1, r, pl.ds(k * 16, 16)] = jnp.zeros((16,), jnp.float32)
        ones_v[r, :] = jnp.full((16,), 1.0, jnp.float32)
        zcnt_v[r, :] = jnp.zeros((16,), jnp.float32)
        return carry
    lax.fori_loop(0, C, zrow, 0)

    for j in range(SPT // C):
        off = s * SPT + j * C
        pltpu.sync_copy(rows_v.at[1], acc_sh.at[pl.ds(off, C)])
        pltpu.sync_copy(zcnt_v, cnt_sh.at[pl.ds(off, C)])
    rem = SPT % C
    if rem:
        off = s * SPT + (SPT // C) * C
        pltpu.sync_copy(rows_v.at[1, pl.ds(0, rem)],
                        acc_sh.at[pl.ds(off, rem)])
        pltpu.sync_copy(zcnt_v.at[pl.ds(0, rem)], cnt_sh.at[pl.ds(off, rem)])
    plsc.subcore_barrier()

    start(1, 1)

    def step(t, carry):
        for b in range(NB):
            g = t * NB + b
            wait(g, b)
            fire(b)
            wait_scat(b)

            @pl.when(g > 0)
            def _():
                wait_ones(b)

            @pl.when(g + 2 < ng)
            def _():
                start(g + 2, b)
        return carry
    lax.fori_loop(0, GLO // NB, step, 0)

    @pl.when(ng > GLO)
    def _():
        wait(GLO, 0)
        fire(0)
        wait_scat(0)
        wait_ones(0)
    wait_ones(0)

    plsc.subcore_barrier()

    pltpu.sync_copy(acc_sh.at[pl.ds(s * SPT, SPT)],
                    sums_hbm.at[c, pl.ds(s * SPT, SPT)])
    pltpu.sync_copy(cnt_sh.at[pl.ds(s * SPT, SPT)],
                    cnt_hbm.at[c, pl.ds(s * SPT, SPT)])


_phase1 = functools.partial(
    pl.kernel,
    out_type=(jax.ShapeDtypeStruct((NC, S, D), jnp.float32),
              jax.ShapeDtypeStruct((NC, S, CW), jnp.float32)),
    mesh=plsc.VectorSubcoreMesh(core_axis_name="c", subcore_axis_name="s",
                                num_cores=NC, num_subcores=NS),
    scratch_types=[
        pltpu.VMEM_SHARED((S, D), jnp.float32),
        pltpu.VMEM_SHARED((S, CW), jnp.float32),
        pltpu.VMEM((NB, C, D), jnp.float32),
        pltpu.VMEM((NB, 1, C), jnp.int32),
        pltpu.VMEM((C, CW), jnp.float32),
        pltpu.VMEM((C, CW), jnp.float32),
    ] + [pltpu.SemaphoreType.DMA] * (3 * NB + 1),
    compiler_params=pltpu.CompilerParams(use_tc_tiling_on_sc=False),
)(_sc_body)


BS = 5000


def _div_body(sums_ref, cnt_ref, out_ref):
    sm = sums_ref[0] + sums_ref[1]
    ct = cnt_ref[0, :, :1] + cnt_ref[1, :, :1]
    out_ref[...] = sm / jnp.maximum(ct, 1.0)


def _phase2(sums, counts):
    return pl.pallas_call(
        _div_body,
        grid=(S // BS,),
        in_specs=[
            pl.BlockSpec((NC, BS, D), lambda i: (0, i, 0)),
            pl.BlockSpec((NC, BS, CW), lambda i: (0, i, 0)),
        ],
        out_specs=pl.BlockSpec((BS, D), lambda i: (i, 0)),
        out_shape=jax.ShapeDtypeStruct((S, D), jnp.float32),
    )(sums, counts)


def kernel(x, segment_ids, num_segments):
    del num_segments
    ids2d = segment_ids.astype(jnp.int32).reshape(NBLK, C)
    sums, counts = _phase1(x, ids2d)
    return _phase2(sums, counts)


# speedup vs baseline: 1.1044x; 1.0331x over previous
import functools

import jax
import jax.numpy as jnp
from jax import lax
from jax.experimental import pallas as pl
from jax.experimental.pallas import tpu as pltpu
from jax.experimental.pallas import tpu_sc as plsc

N = 320000
D = 128
S = 10000
NC = 2
NS = 16
NW = NC * NS
RPT = N // NW
C = 80
G = RPT // C
NB = 3
SPT = S // NS
CW = 16


def _sc_body(x_hbm, ids_hbm, sums_hbm, cnt_hbm,
             acc_sh, cnt_sh, rows_v, ids_v, ones_v, zcnt_v, *sems):
    c = lax.axis_index("c")
    s = lax.axis_index("s")
    wid = c * NS + s
    row0 = wid * RPT

    gsems = sems[0:NB]
    isems = sems[NB:2 * NB]
    ssems = sems[2 * NB:3 * NB]
    osem = sems[3 * NB]

    def start(g, b):
        pltpu.async_copy(x_hbm.at[pl.ds(row0 + g * C, C)], rows_v.at[b],
                         gsems[b])
        pltpu.async_copy(ids_hbm.at[pl.ds(row0 + g * C, C)], ids_v.at[b, 0],
                         isems[b])

    def wait(g, b):
        pltpu.make_async_copy(x_hbm.at[pl.ds(row0 + g * C, C)],
                              rows_v.at[b], gsems[b]).wait()
        pltpu.make_async_copy(ids_hbm.at[pl.ds(row0 + g * C, C)],
                              ids_v.at[b, 0], isems[b]).wait()

    def fire(b):
        idx = ids_v.at[b, 0]
        pltpu.async_copy(rows_v.at[b], acc_sh.at[idx], ssems[b], add=True)
        pltpu.async_copy(ones_v, cnt_sh.at[idx], osem, add=True)

    def wait_scat(b):
        idx = ids_v.at[b, 0]
        pltpu.make_async_copy(rows_v.at[b], acc_sh.at[idx], ssems[b]).wait()

    def wait_ones(b):
        idx = ids_v.at[b, 0]
        pltpu.make_async_copy(ones_v, cnt_sh.at[idx], osem).wait()

    start(0, 0)

    def zrow(r, carry):
        for k in range(D // 16):
            rows_v[NB - 1, r, pl.ds(k * 16, 16)] = jnp.zeros((16,),
                                                             jnp.float32)
        ones_v[r, :] = jnp.full((16,), 1.0, jnp.float32)
        zcnt_v[r, :] = jnp.zeros((16,), jnp.float32)
        return carry
    lax.fori_loop(0, C, zrow, 0)

    for j in range(SPT // C):
        off = s * SPT + j * C
        pltpu.sync_copy(rows_v.at[NB - 1], acc_sh.at[pl.ds(off, C)])
        pltpu.sync_copy(zcnt_v, cnt_sh.at[pl.ds(off, C)])
    rem = SPT % C
    if rem:
        off = s * SPT + (SPT // C) * C
        pltpu.sync_copy(rows_v.at[NB - 1, pl.ds(0, rem)],
                        acc_sh.at[pl.ds(off, rem)])
        pltpu.sync_copy(zcnt_v.at[pl.ds(0, rem)], cnt_sh.at[pl.ds(off, rem)])
    plsc.subcore_barrier()

    start(1, 1)

    def body(g, b):
        wait(g, b)
        fire(b)

        @pl.when(g >= 1)
        def _():
            wait_ones(b)
            wait_scat((b + 2) % NB)

        @pl.when(g + 2 < G)
        def _():
            start(g + 2, (b + 2) % NB)

    def step(t, carry):
        for b in range(NB):
            body(t * NB + b, b)
        return carry
    lax.fori_loop(0, G // NB, step, 0)
    for g in range((G // NB) * NB, G):
        b = g % NB
        wait(g, b)
        fire(b)
        wait_ones(b)
        wait_scat((b + 2) % NB)
    wait_scat((G - 1) % NB)
    wait_ones(0)

    plsc.subcore_barrier()

    pltpu.sync_copy(acc_sh.at[pl.ds(s * SPT, SPT)],
                    sums_hbm.at[c, pl.ds(s * SPT, SPT)])
    pltpu.sync_copy(cnt_sh.at[pl.ds(s * SPT, SPT)],
                    cnt_hbm.at[c, pl.ds(s * SPT, SPT)])


_phase1 = functools.partial(
    pl.kernel,
    out_type=(jax.ShapeDtypeStruct((NC, S, D), jnp.float32),
              jax.ShapeDtypeStruct((NC, S, CW), jnp.float32)),
    mesh=plsc.VectorSubcoreMesh(core_axis_name="c", subcore_axis_name="s",
                                num_cores=NC, num_subcores=NS),
    scratch_types=[
        pltpu.VMEM_SHARED((S, D), jnp.float32),
        pltpu.VMEM_SHARED((S, CW), jnp.float32),
        pltpu.VMEM((NB, C, D), jnp.float32),
        pltpu.VMEM((NB, 1, C), jnp.int32),
        pltpu.VMEM((C, CW), jnp.float32),
        pltpu.VMEM((C, CW), jnp.float32),
    ] + [pltpu.SemaphoreType.DMA] * (3 * NB + 1),
    compiler_params=pltpu.CompilerParams(use_tc_tiling_on_sc=False),
)(_sc_body)


BS = 5000


def _div_body(sums_ref, cnt_ref, out_ref):
    sm = sums_ref[0] + sums_ref[1]
    ct = cnt_ref[0, :, :1] + cnt_ref[1, :, :1]
    out_ref[...] = sm / jnp.maximum(ct, 1.0)


def _phase2(sums, counts):
    return pl.pallas_call(
        _div_body,
        grid=(S // BS,),
        in_specs=[
            pl.BlockSpec((NC, BS, D), lambda i: (0, i, 0)),
            pl.BlockSpec((NC, BS, CW), lambda i: (0, i, 0)),
        ],
        out_specs=pl.BlockSpec((BS, D), lambda i: (i, 0)),
        out_shape=jax.ShapeDtypeStruct((S, D), jnp.float32),
    )(sums, counts)


def kernel(x, segment_ids, num_segments):
    del num_segments
    sums, counts = _phase1(x, segment_ids.astype(jnp.int32))
    return _phase2(sums, counts)
